# baseline (speedup 1.0000x reference)
import functools

import jax
import jax.numpy as jnp
from jax import lax
from jax.experimental import pallas as pl
from jax.experimental.pallas import tpu as pltpu
from jax.experimental.pallas import tpu_sc as plsc

N = 50000
E = 800000
NCOLS = 4
EDIM = 16
PROJ = 64
VOCAB = 100000
OUTC = 32
HALF = PROJ // 2

NC = 2
NS = 16
NW = NC * NS

CH = 100
ROWS_PER_SUB = E // CH // NS
STRIPE = N // NS
ZROWS = 625

EMB_PER_W = 6400
EMB_OUT = NW * EMB_PER_W
EMB_K = 8


def _mesh():
    return plsc.VectorSubcoreMesh(core_axis_name="c", subcore_axis_name="s")


_SC_PARAMS = pltpu.CompilerParams(use_tc_tiling_on_sc=False)


def _zero_fill(ref, nrows, width):
    @pl.loop(0, nrows)
    def _(r):
        for c0 in range(0, width, 16):
            ref[r, pl.ds(c0, 16)] = jnp.zeros((16,), jnp.float32)


def _sc_embed(tab_u, idx_u, tab_i, idx_i):
    out_t = (jax.ShapeDtypeStruct((EMB_OUT, EDIM), jnp.float32),) * 2

    @functools.partial(
        pl.kernel, out_type=out_t, mesh=_mesh(),
        compiler_params=_SC_PARAMS,
        scratch_types=[pltpu.VMEM((EMB_K, CH), jnp.int32),
                       pltpu.VMEM((EMB_K * CH, EDIM), jnp.float32),
                       pltpu.SemaphoreType.DMA])
    def k(tu, iu, ti, ii, ou, oi, idx_v, rows_v, sem):
        w = lax.axis_index("s") * NC + lax.axis_index("c")

        def run(tab, ih, oh):
            @pl.loop(0, EMB_PER_W // (EMB_K * CH))
            def _(o):
                r0 = w * (EMB_PER_W // CH) + o * EMB_K
                pltpu.sync_copy(ih.at[pl.ds(r0, EMB_K)], idx_v)
                cps = [pltpu.async_copy(tab.at[idx_v.at[j]],
                                        rows_v.at[pl.ds(j * CH, CH)], sem)
                       for j in range(EMB_K)]
                for cp in cps:
                    cp.wait()
                pltpu.sync_copy(rows_v, oh.at[pl.ds(r0 * CH, EMB_K * CH)])

        run(tu, iu, ou)
        run(ti, ii, oi)

    return k(tab_u, idx_u, tab_i, idx_i)


def _sc_counts(d0, d1, d2, d3):
    out_t = jax.ShapeDtypeStruct((4, NC, N, 16), jnp.float32)

    @functools.partial(
        pl.kernel, out_type=out_t, mesh=_mesh(),
        compiler_params=_SC_PARAMS,
        scratch_types=[pltpu.VMEM((20, CH), jnp.int32),
                       pltpu.VMEM((CH, 16), jnp.float32),
                       pltpu.VMEM((ZROWS, 16), jnp.float32),
                       pltpu.VMEM_SHARED((N, 16), jnp.float32),
                       pltpu.SemaphoreType.DMA])
    def k(dh0, dh1, dh2, dh3, oh, idx_v, ones_v, zb_v, acc, sem):
        c = lax.axis_index("c")
        s = lax.axis_index("s")
        _zero_fill(zb_v, ZROWS, 16)

        @pl.loop(0, CH)
        def _(r):
            ones_v[r, pl.ds(0, 16)] = jnp.ones((16,), jnp.float32)

        base = s * STRIPE

        def one_type(dh, oc):
            @pl.loop(0, STRIPE // ZROWS)
            def _(kk):
                pltpu.sync_copy(zb_v, acc.at[pl.ds(base + kk * ZROWS, ZROWS)])
            plsc.subcore_barrier()
            row0 = c * 4000 + s * 250

            def fire(r0, nrow):
                pltpu.sync_copy(dh.at[pl.ds(r0, nrow)],
                                idx_v.at[pl.ds(0, nrow)])
                cps = [pltpu.async_copy(ones_v, acc.at[idx_v.at[j]], sem,
                                        add=True)
                       for j in range(nrow)]
                for cp in cps:
                    cp.wait()

            @pl.loop(0, 12)
            def _(o):
                fire(row0 + o * 20, 20)
            fire(row0 + 240, 10)
            plsc.subcore_barrier()
            pltpu.sync_copy(acc.at[pl.ds(base, STRIPE)],
                            oc.at[pl.ds(base, STRIPE)])
            plsc.subcore_barrier()

        for t, dh in enumerate((dh0, dh1, dh2, dh3)):
            @pl.when(c == 0)
            def _(dh=dh, t=t):
                one_type(dh, oh.at[t].at[0])

            @pl.when(c == 1)
            def _(dh=dh, t=t):
                one_type(dh, oh.at[t].at[1])

    return k(d0, d1, d2, d3)


def _sc_agg(h_split, src2d, dst2d):
    out_t = jax.ShapeDtypeStruct((NC, N, HALF), jnp.float32)

    @functools.partial(
        pl.kernel, out_type=out_t, mesh=_mesh(),
        compiler_params=_SC_PARAMS,
        scratch_types=[pltpu.VMEM((8, CH), jnp.int32),
                       pltpu.VMEM((8, CH), jnp.int32),
                       pltpu.VMEM((8 * CH, HALF), jnp.float32),
                       pltpu.VMEM_SHARED((N, HALF), jnp.float32),
                       pltpu.SemaphoreType.DMA,
                       pltpu.SemaphoreType.DMA])
    def k(hh, sh, dh, oh, sidx, didx, rows, acc, sem_g, sem_s):
        c = lax.axis_index("c")
        s = lax.axis_index("s")
        base = s * STRIPE
        _zero_fill(rows, ZROWS, HALF)

        @pl.loop(0, STRIPE // ZROWS)
        def _(kk):
            pltpu.sync_copy(rows.at[pl.ds(0, ZROWS)],
                            acc.at[pl.ds(base + kk * ZROWS, ZROWS)])
        plsc.subcore_barrier()

        def run(tab, oc):
            row0 = s * ROWS_PER_SUB
            HB = 4

            def load_idx(h, c):
                pltpu.sync_copy(sh.at[pl.ds(row0 + c * HB, HB)],
                                sidx.at[pl.ds(h * HB, HB)])
                pltpu.sync_copy(dh.at[pl.ds(row0 + c * HB, HB)],
                                didx.at[pl.ds(h * HB, HB)])

            def gathers(h):
                return [pltpu.async_copy(tab.at[sidx.at[h * HB + j]],
                                         rows.at[pl.ds((h * HB + j) * CH, CH)],
                                         sem_g)
                        for j in range(HB)]

            def wait_gathers(h):
                for j in range(HB):
                    pltpu.make_async_copy(
                        tab.at[sidx.at[h * HB + j]],
                        rows.at[pl.ds((h * HB + j) * CH, CH)],
                        sem_g).wait()

            def scatters(h):
                return [pltpu.async_copy(rows.at[pl.ds((h * HB + j) * CH, CH)],
                                         acc.at[didx.at[h * HB + j]],
                                         sem_s, add=True)
                        for j in range(HB)]

            def wait_scatters(h):
                for j in range(HB):
                    pltpu.make_async_copy(
                        rows.at[pl.ds((h * HB + j) * CH, CH)],
                        acc.at[didx.at[h * HB + j]],
                        sem_s).wait()

            load_idx(0, 0)
            gathers(0)
            load_idx(1, 1)
            gathers(1)
            wait_gathers(0)
            scatters(0)

            @pl.loop(0, 61)
            def _(kk):
                c = 2 * kk
                wait_gathers(1)
                scatters(1)
                wait_scatters(0)
                load_idx(0, c + 2)
                gathers(0)
                wait_gathers(0)
                scatters(0)
                wait_scatters(1)
                load_idx(1, c + 3)
                gathers(1)

            wait_gathers(1)
            scatters(1)
            wait_scatters(0)
            load_idx(0, 124)
            gathers(0)
            wait_gathers(0)
            scatters(0)
            wait_scatters(1)
            wait_scatters(0)
            plsc.subcore_barrier()
            pltpu.sync_copy(acc.at[pl.ds(base, STRIPE)],
                            oc.at[pl.ds(base, STRIPE)])

        @pl.when(c == 0)
        def _():
            run(hh.at[0], oh.at[0])

        @pl.when(c == 1)
        def _():
            run(hh.at[1], oh.at[1])

    return k(h_split, src2d, dst2d)


R = 1000
G = N // R


def _row_spec(shape):
    nd = len(shape)
    if nd == 2:
        return pl.BlockSpec((R, shape[1]), lambda i: (i, 0))
    return pl.BlockSpec((shape[0], R, shape[2]), lambda i: (0, i, 0))


def _full_spec(shape):
    return pl.BlockSpec(shape, lambda i: (0,) * len(shape))


def _split_out():
    return (jax.ShapeDtypeStruct((NC, N, HALF), jnp.float32),
            pl.BlockSpec((NC, R, HALF), lambda i: (0, i, 0)))


def _proj_body(gu_ref, gi_ref, wu_ref, bu_ref, wi_ref, bi_ref, ou_ref, oi_ref):
    for g_ref, w_ref, b_ref, o_ref in ((gu_ref, wu_ref, bu_ref, ou_ref),
                                       (gi_ref, wi_ref, bi_ref, oi_ref)):
        h = jnp.dot(g_ref[...], w_ref[...],
                    preferred_element_type=jnp.float32) + b_ref[...][0:1, :]
        o_ref[0] = h[:, :HALF]
        o_ref[1] = h[:, HALF:]


def _tc_proj(gu, gi, wu, bu, wi, bi):
    outs = (_split_out(), _split_out())
    return pl.pallas_call(
        _proj_body,
        grid=(G,),
        in_specs=[_row_spec(gu.shape), _row_spec(gi.shape),
                  _full_spec(wu.shape), _full_spec(bu.shape),
                  _full_spec(wi.shape), _full_spec(bi.shape)],
        out_shape=tuple(o[0] for o in outs),
        out_specs=tuple(o[1] for o in outs),
    )(gu, gi, wu, bu, wi, bi)


def _layer_body(has_ht, *refs):
    if has_ht:
        (a0, a1, a2, a3, c0, c1, c2, c3, hu, hi, ht,
         wl, wr, wb, ou, oi, ot) = refs
    else:
        (a0, a1, a2, a3, c0, c1, c2, c3, hu, hi,
         wl, wr, wb, ou, oi, ot) = refs

    def full(ref):
        v = ref[...]
        return jnp.concatenate([v[0], v[1]], axis=1)

    def mean(aref, cref):
        cv = cref[...]
        cnt = cv[0, :, 0] + cv[1, :, 0]
        inv = 1.0 / jnp.maximum(cnt, 1.0)
        return full(aref) * inv[:, None]

    wlv = wl[...]
    wrv = wr[...]
    wbv = wb[...]
    dot = lambda x, w: jnp.dot(x, w, preferred_element_type=jnp.float32)
    m0, m1, m2, m3 = mean(a0, c0), mean(a1, c1), mean(a2, c2), mean(a3, c3)
    hu_v = full(hu)
    hi_v = full(hi)
    yi = dot(m0, wlv[0]) + dot(hi_v, wrv[0]) + wbv[0, 0:1, :]
    yu = dot(m1, wlv[1]) + dot(hu_v, wrv[1]) + wbv[1, 0:1, :]
    if has_ht:
        ht_v = full(ht)
        t2 = dot(m2, wlv[2]) + dot(ht_v, wrv[2]) + wbv[2, 0:1, :]
        t3 = dot(m3, wlv[3]) + dot(ht_v, wrv[3]) + wbv[3, 0:1, :]
    else:
        t2 = dot(m2, wlv[2]) + jnp.sum(wrv[2], axis=0)[None, :] + wbv[2, 0:1, :]
        t3 = dot(m3, wlv[3]) + jnp.sum(wrv[3], axis=0)[None, :] + wbv[3, 0:1, :]
    yt = 0.5 * (t2 + t3)
    for o_ref, y in ((ou, yu), (oi, yi), (ot, yt)):
        o_ref[0] = y[:, :HALF]
        o_ref[1] = y[:, HALF:]


def _tc_layer(aggs, cnts, hu, hi, ht, wl, wr, wb):
    has_ht = ht is not None
    ops = list(aggs) + list(cnts) + [hu, hi] + ([ht] if has_ht else []) \
        + [wl, wr, wb]
    in_specs = [_row_spec(a.shape) for a in aggs] \
        + [_row_spec(c.shape) for c in cnts] \
        + [_row_spec(hu.shape), _row_spec(hi.shape)] \
        + ([_row_spec(ht.shape)] if has_ht else []) \
        + [_full_spec(wl.shape), _full_spec(wr.shape), _full_spec(wb.shape)]
    outs = (_split_out(), _split_out(), _split_out())
    return pl.pallas_call(
        functools.partial(_layer_body, has_ht),
        grid=(G,),
        in_specs=in_specs,
        out_shape=tuple(o[0] for o in outs),
        out_specs=tuple(o[1] for o in outs),
    )(*ops)


def _last_body(a2, a3, c2, c3, ht, wl, wr, wb, ow, ob, o_ref):
    def full(ref):
        v = ref[...]
        return jnp.concatenate([v[0], v[1]], axis=1)

    def mean(aref, cref):
        cv = cref[...]
        cnt = cv[0, :, 0] + cv[1, :, 0]
        inv = 1.0 / jnp.maximum(cnt, 1.0)
        return full(aref) * inv[:, None]

    dot = lambda x, w: jnp.dot(x, w, preferred_element_type=jnp.float32)
    wlv, wrv, wbv = wl[...], wr[...], wb[...]
    ht_v = full(ht)
    t2 = dot(mean(a2, c2), wlv[0]) + dot(ht_v, wrv[0]) + wbv[0, 0:1, :]
    t3 = dot(mean(a3, c3), wlv[1]) + dot(ht_v, wrv[1]) + wbv[1, 0:1, :]
    yt = 0.5 * (t2 + t3)
    lg = dot(yt, ow[...]) + ob[...][0:1, :]
    m = jnp.max(lg, axis=1, keepdims=True)
    e = jnp.exp(lg - m)
    o_ref[...] = e / jnp.sum(e, axis=1, keepdims=True)


def _tc_last(a2, a3, c2, c3, ht, wl, wr, wb, ow, ob):
    ops = [a2, a3, c2, c3, ht, wl, wr, wb, ow, ob]
    in_specs = [_row_spec(a2.shape), _row_spec(a3.shape),
                _row_spec(c2.shape), _row_spec(c3.shape),
                _row_spec(ht.shape),
                _full_spec(wl.shape), _full_spec(wr.shape),
                _full_spec(wb.shape), _full_spec(ow.shape),
                _full_spec(ob.shape)]
    return pl.pallas_call(
        _last_body,
        grid=(G,),
        in_specs=in_specs,
        out_shape=jax.ShapeDtypeStruct((N, OUTC), jnp.float32),
        out_specs=pl.BlockSpec((R, OUTC), lambda i: (i, 0)),
    )(*ops)


def kernel(x_users, x_items, x_target, ei_u2i, ei_i2u, ei_i2t, ei_u2t,
           emb_users, emb_items, Wp_u, bp_u, Wp_i, bp_i, Wl, Wr, Wb,
           out_W, out_b):
    i32 = lambda a: a.astype(jnp.int32)

    offs = (jnp.arange(NCOLS, dtype=jnp.int32) * VOCAB)[None, :]
    pad = jnp.zeros((EMB_OUT - N * NCOLS,), jnp.int32)
    iu = jnp.concatenate([(i32(x_users) + offs).reshape(-1), pad])
    ii = jnp.concatenate([(i32(x_items) + offs).reshape(-1), pad])
    iu = iu.reshape(EMB_OUT // CH, CH)
    ii = ii.reshape(EMB_OUT // CH, CH)
    tab_u = emb_users.reshape(NCOLS * VOCAB, EDIM)
    tab_i = emb_items.reshape(NCOLS * VOCAB, EDIM)

    edges = []
    for ei in (ei_u2i, ei_i2u, ei_i2t, ei_u2t):
        e2 = i32(ei).reshape(2, E // CH, CH)
        edges.append((e2[0], e2[1]))

    gu_raw, gi_raw = _sc_embed(tab_u, iu, tab_i, ii)
    gu = gu_raw.reshape(EMB_OUT // PROJ * EDIM, PROJ)
    gi = gi_raw.reshape(EMB_OUT // PROJ * EDIM, PROJ)

    cnt_all = _sc_counts(edges[0][1], edges[1][1], edges[2][1], edges[3][1])
    cnts = [cnt_all[t] for t in range(4)]

    tile8 = lambda v: jnp.tile(v[None, :], (8, 1))
    hu, hi = _tc_proj(gu, gi, Wp_u, tile8(bp_u), Wp_i, tile8(bp_i))

    srcs = (hu, hi, hi, hu)
    aggs = [_sc_agg(srcs[t], edges[t][0], edges[t][1]) for t in range(4)]
    wb_0 = jnp.tile(Wb[0][:, None, :], (1, 8, 1))
    hu, hi, ht = _tc_layer(aggs, cnts, hu, hi, None, Wl[0], Wr[0], wb_0)

    a2 = _sc_agg(hi, edges[2][0], edges[2][1])
    a3 = _sc_agg(hu, edges[3][0], edges[3][1])
    wb_1 = jnp.tile(Wb[1, 2:4][:, None, :], (1, 8, 1))
    return _tc_last(a2, a3, cnts[2], cnts[3], ht,
                    Wl[1, 2:4], Wr[1, 2:4], wb_1, out_W, tile8(out_b))


# speedup vs baseline: 1.0935x; 1.0935x over previous
import functools

import jax
import jax.numpy as jnp
from jax import lax
from jax.experimental import pallas as pl
from jax.experimental.pallas import tpu as pltpu
from jax.experimental.pallas import tpu_sc as plsc

N = 50000
E = 800000
NCOLS = 4
EDIM = 16
PROJ = 64
VOCAB = 100000
OUTC = 32
HALF = PROJ // 2

NC = 2
NS = 16
NW = NC * NS

CH = 100
ROWS_PER_SUB = E // CH // NS
STRIPE = N // NS
ZROWS = 625

EMB_PER_W = 6400
EMB_OUT = NW * EMB_PER_W
EMB_K = 8


def _mesh():
    return plsc.VectorSubcoreMesh(core_axis_name="c", subcore_axis_name="s")


_SC_PARAMS = pltpu.CompilerParams(use_tc_tiling_on_sc=False)


def _zero_fill(ref, nrows, width):
    @pl.loop(0, nrows)
    def _(r):
        for c0 in range(0, width, 16):
            ref[r, pl.ds(c0, 16)] = jnp.zeros((16,), jnp.float32)


def _sc_embed(tab_u, idx_u, tab_i, idx_i):
    out_t = (jax.ShapeDtypeStruct((EMB_OUT, EDIM), jnp.float32),) * 2

    @functools.partial(
        pl.kernel, out_type=out_t, mesh=_mesh(),
        compiler_params=_SC_PARAMS,
        scratch_types=[pltpu.VMEM((EMB_K, CH), jnp.int32),
                       pltpu.VMEM((EMB_K * CH, EDIM), jnp.float32),
                       pltpu.SemaphoreType.DMA])
    def k(tu, iu, ti, ii, ou, oi, idx_v, rows_v, sem):
        w = lax.axis_index("s") * NC + lax.axis_index("c")

        def run(tab, ih, oh):
            @pl.loop(0, EMB_PER_W // (EMB_K * CH))
            def _(o):
                r0 = w * (EMB_PER_W // CH) + o * EMB_K
                pltpu.sync_copy(ih.at[pl.ds(r0, EMB_K)], idx_v)
                cps = [pltpu.async_copy(tab.at[idx_v.at[j]],
                                        rows_v.at[pl.ds(j * CH, CH)], sem)
                       for j in range(EMB_K)]
                for cp in cps:
                    cp.wait()
                pltpu.sync_copy(rows_v, oh.at[pl.ds(r0 * CH, EMB_K * CH)])

        run(tu, iu, ou)
        run(ti, ii, oi)

    return k(tab_u, idx_u, tab_i, idx_i)


def _sc_counts(d0, d1, d2, d3):
    out_t = jax.ShapeDtypeStruct((4, NC, N, 16), jnp.float32)

    @functools.partial(
        pl.kernel, out_type=out_t, mesh=_mesh(),
        compiler_params=_SC_PARAMS,
        scratch_types=[pltpu.VMEM((20, CH), jnp.int32),
                       pltpu.VMEM((CH, 16), jnp.float32),
                       pltpu.VMEM((ZROWS, 16), jnp.float32),
                       pltpu.VMEM_SHARED((N, 16), jnp.float32),
                       pltpu.SemaphoreType.DMA])
    def k(dh0, dh1, dh2, dh3, oh, idx_v, ones_v, zb_v, acc, sem):
        c = lax.axis_index("c")
        s = lax.axis_index("s")
        _zero_fill(zb_v, ZROWS, 16)

        @pl.loop(0, CH)
        def _(r):
            ones_v[r, pl.ds(0, 16)] = jnp.ones((16,), jnp.float32)

        base = s * STRIPE

        def one_type(dh, oc):
            @pl.loop(0, STRIPE // ZROWS)
            def _(kk):
                pltpu.sync_copy(zb_v, acc.at[pl.ds(base + kk * ZROWS, ZROWS)])
            plsc.subcore_barrier()
            row0 = c * 4000 + s * 250

            def fire(r0, nrow):
                pltpu.sync_copy(dh.at[pl.ds(r0, nrow)],
                                idx_v.at[pl.ds(0, nrow)])
                cps = [pltpu.async_copy(ones_v, acc.at[idx_v.at[j]], sem,
                                        add=True)
                       for j in range(nrow)]
                for cp in cps:
                    cp.wait()

            @pl.loop(0, 12)
            def _(o):
                fire(row0 + o * 20, 20)
            fire(row0 + 240, 10)
            plsc.subcore_barrier()
            pltpu.sync_copy(acc.at[pl.ds(base, STRIPE)],
                            oc.at[pl.ds(base, STRIPE)])
            plsc.subcore_barrier()

        for t, dh in enumerate((dh0, dh1, dh2, dh3)):
            @pl.when(c == 0)
            def _(dh=dh, t=t):
                one_type(dh, oh.at[t].at[0])

            @pl.when(c == 1)
            def _(dh=dh, t=t):
                one_type(dh, oh.at[t].at[1])

    return k(d0, d1, d2, d3)


def _sc_agg(h_split, src2d, dst2d):
    out_t = jax.ShapeDtypeStruct((NC, N, HALF), jnp.float32)

    @functools.partial(
        pl.kernel, out_type=out_t, mesh=_mesh(),
        compiler_params=_SC_PARAMS,
        scratch_types=[pltpu.VMEM((8, CH), jnp.int32),
                       pltpu.VMEM((8, CH), jnp.int32),
                       pltpu.VMEM((8 * CH, HALF), jnp.float32),
                       pltpu.VMEM_SHARED((N, HALF), jnp.float32),
                       pltpu.SemaphoreType.DMA,
                       pltpu.SemaphoreType.DMA])
    def k(hh, sh, dh, oh, sidx, didx, rows, acc, sem_g, sem_s):
        c = lax.axis_index("c")
        s = lax.axis_index("s")
        base = s * STRIPE
        _zero_fill(rows, ZROWS, HALF)

        @pl.loop(0, STRIPE // ZROWS)
        def _(kk):
            pltpu.sync_copy(rows.at[pl.ds(0, ZROWS)],
                            acc.at[pl.ds(base + kk * ZROWS, ZROWS)])
        plsc.subcore_barrier()

        def run(tab, oc):
            row0 = s * ROWS_PER_SUB

            def step(r0, nrow):
                pltpu.sync_copy(sh.at[pl.ds(r0, nrow)],
                                sidx.at[pl.ds(0, nrow)])
                pltpu.sync_copy(dh.at[pl.ds(r0, nrow)],
                                didx.at[pl.ds(0, nrow)])
                gs = [pltpu.async_copy(tab.at[sidx.at[j]],
                                       rows.at[pl.ds(j * CH, CH)], sem_g)
                      for j in range(nrow)]
                for cp in gs:
                    cp.wait()
                ss = [pltpu.async_copy(rows.at[pl.ds(j * CH, CH)],
                                       acc.at[didx.at[j]], sem_s, add=True)
                      for j in range(nrow)]
                for cp in ss:
                    cp.wait()

            @pl.loop(0, 62)
            def _(o):
                step(row0 + o * 8, 8)
            step(row0 + 496, 4)
            plsc.subcore_barrier()
            pltpu.sync_copy(acc.at[pl.ds(base, STRIPE)],
                            oc.at[pl.ds(base, STRIPE)])

        @pl.when(c == 0)
        def _():
            run(hh.at[0], oh.at[0])

        @pl.when(c == 1)
        def _():
            run(hh.at[1], oh.at[1])

    return k(h_split, src2d, dst2d)


R = 1000
G = N // R


def _row_spec(shape):
    nd = len(shape)
    if nd == 2:
        return pl.BlockSpec((R, shape[1]), lambda i: (i, 0))
    return pl.BlockSpec((shape[0], R, shape[2]), lambda i: (0, i, 0))


def _full_spec(shape):
    return pl.BlockSpec(shape, lambda i: (0,) * len(shape))


def _split_out():
    return (jax.ShapeDtypeStruct((NC, N, HALF), jnp.float32),
            pl.BlockSpec((NC, R, HALF), lambda i: (0, i, 0)))


def _proj_body(gu_ref, gi_ref, wu_ref, bu_ref, wi_ref, bi_ref, ou_ref, oi_ref):
    for g_ref, w_ref, b_ref, o_ref in ((gu_ref, wu_ref, bu_ref, ou_ref),
                                       (gi_ref, wi_ref, bi_ref, oi_ref)):
        h = jnp.dot(g_ref[...], w_ref[...],
                    preferred_element_type=jnp.float32) + b_ref[...][0:1, :]
        o_ref[0] = h[:, :HALF]
        o_ref[1] = h[:, HALF:]


def _tc_proj(gu, gi, wu, bu, wi, bi):
    outs = (_split_out(), _split_out())
    return pl.pallas_call(
        _proj_body,
        grid=(G,),
        in_specs=[_row_spec(gu.shape), _row_spec(gi.shape),
                  _full_spec(wu.shape), _full_spec(bu.shape),
                  _full_spec(wi.shape), _full_spec(bi.shape)],
        out_shape=tuple(o[0] for o in outs),
        out_specs=tuple(o[1] for o in outs),
    )(gu, gi, wu, bu, wi, bi)


def _layer_body(has_ht, *refs):
    if has_ht:
        (a0, a1, a2, a3, c0, c1, c2, c3, hu, hi, ht,
         wl, wr, wb, ou, oi, ot) = refs
    else:
        (a0, a1, a2, a3, c0, c1, c2, c3, hu, hi,
         wl, wr, wb, ou, oi, ot) = refs

    def full(ref):
        v = ref[...]
        return jnp.concatenate([v[0], v[1]], axis=1)

    def mean(aref, cref):
        cv = cref[...]
        cnt = cv[0, :, 0] + cv[1, :, 0]
        inv = 1.0 / jnp.maximum(cnt, 1.0)
        return full(aref) * inv[:, None]

    wlv = wl[...]
    wrv = wr[...]
    wbv = wb[...]
    dot = lambda x, w: jnp.dot(x, w, preferred_element_type=jnp.float32)
    m0, m1, m2, m3 = mean(a0, c0), mean(a1, c1), mean(a2, c2), mean(a3, c3)
    hu_v = full(hu)
    hi_v = full(hi)
    yi = dot(m0, wlv[0]) + dot(hi_v, wrv[0]) + wbv[0, 0:1, :]
    yu = dot(m1, wlv[1]) + dot(hu_v, wrv[1]) + wbv[1, 0:1, :]
    if has_ht:
        ht_v = full(ht)
        t2 = dot(m2, wlv[2]) + dot(ht_v, wrv[2]) + wbv[2, 0:1, :]
        t3 = dot(m3, wlv[3]) + dot(ht_v, wrv[3]) + wbv[3, 0:1, :]
    else:
        t2 = dot(m2, wlv[2]) + jnp.sum(wrv[2], axis=0)[None, :] + wbv[2, 0:1, :]
        t3 = dot(m3, wlv[3]) + jnp.sum(wrv[3], axis=0)[None, :] + wbv[3, 0:1, :]
    yt = 0.5 * (t2 + t3)
    for o_ref, y in ((ou, yu), (oi, yi), (ot, yt)):
        o_ref[0] = y[:, :HALF]
        o_ref[1] = y[:, HALF:]


def _tc_layer(aggs, cnts, hu, hi, ht, wl, wr, wb):
    has_ht = ht is not None
    ops = list(aggs) + list(cnts) + [hu, hi] + ([ht] if has_ht else []) \
        + [wl, wr, wb]
    in_specs = [_row_spec(a.shape) for a in aggs] \
        + [_row_spec(c.shape) for c in cnts] \
        + [_row_spec(hu.shape), _row_spec(hi.shape)] \
        + ([_row_spec(ht.shape)] if has_ht else []) \
        + [_full_spec(wl.shape), _full_spec(wr.shape), _full_spec(wb.shape)]
    outs = (_split_out(), _split_out(), _split_out())
    return pl.pallas_call(
        functools.partial(_layer_body, has_ht),
        grid=(G,),
        in_specs=in_specs,
        out_shape=tuple(o[0] for o in outs),
        out_specs=tuple(o[1] for o in outs),
    )(*ops)


def _last_body(a2, a3, c2, c3, ht, wl, wr, wb, ow, ob, o_ref):
    def full(ref):
        v = ref[...]
        return jnp.concatenate([v[0], v[1]], axis=1)

    def mean(aref, cref):
        cv = cref[...]
        cnt = cv[0, :, 0] + cv[1, :, 0]
        inv = 1.0 / jnp.maximum(cnt, 1.0)
        return full(aref) * inv[:, None]

    dot = lambda x, w: jnp.dot(x, w, preferred_element_type=jnp.float32)
    wlv, wrv, wbv = wl[...], wr[...], wb[...]
    ht_v = full(ht)
    t2 = dot(mean(a2, c2), wlv[0]) + dot(ht_v, wrv[0]) + wbv[0, 0:1, :]
    t3 = dot(mean(a3, c3), wlv[1]) + dot(ht_v, wrv[1]) + wbv[1, 0:1, :]
    yt = 0.5 * (t2 + t3)
    lg = dot(yt, ow[...]) + ob[...][0:1, :]
    m = jnp.max(lg, axis=1, keepdims=True)
    e = jnp.exp(lg - m)
    o_ref[...] = e / jnp.sum(e, axis=1, keepdims=True)


def _tc_last(a2, a3, c2, c3, ht, wl, wr, wb, ow, ob):
    ops = [a2, a3, c2, c3, ht, wl, wr, wb, ow, ob]
    in_specs = [_row_spec(a2.shape), _row_spec(a3.shape),
                _row_spec(c2.shape), _row_spec(c3.shape),
                _row_spec(ht.shape),
                _full_spec(wl.shape), _full_spec(wr.shape),
                _full_spec(wb.shape), _full_spec(ow.shape),
                _full_spec(ob.shape)]
    return pl.pallas_call(
        _last_body,
        grid=(G,),
        in_specs=in_specs,
        out_shape=jax.ShapeDtypeStruct((N, OUTC), jnp.float32),
        out_specs=pl.BlockSpec((R, OUTC), lambda i: (i, 0)),
    )(*ops)


def kernel(x_users, x_items, x_target, ei_u2i, ei_i2u, ei_i2t, ei_u2t,
           emb_users, emb_items, Wp_u, bp_u, Wp_i, bp_i, Wl, Wr, Wb,
           out_W, out_b):
    i32 = lambda a: a.astype(jnp.int32)

    offs = (jnp.arange(NCOLS, dtype=jnp.int32) * VOCAB)[None, :]
    pad = jnp.zeros((EMB_OUT - N * NCOLS,), jnp.int32)
    iu = jnp.concatenate([(i32(x_users) + offs).reshape(-1), pad])
    ii = jnp.concatenate([(i32(x_items) + offs).reshape(-1), pad])
    iu = iu.reshape(EMB_OUT // CH, CH)
    ii = ii.reshape(EMB_OUT // CH, CH)
    tab_u = emb_users.reshape(NCOLS * VOCAB, EDIM)
    tab_i = emb_items.reshape(NCOLS * VOCAB, EDIM)

    edges = []
    for ei in (ei_u2i, ei_i2u, ei_i2t, ei_u2t):
        e2 = i32(ei).reshape(2, E // CH, CH)
        edges.append((e2[0], e2[1]))

    gu_raw, gi_raw = _sc_embed(tab_u, iu, tab_i, ii)
    gu = gu_raw.reshape(EMB_OUT // PROJ * EDIM, PROJ)
    gi = gi_raw.reshape(EMB_OUT // PROJ * EDIM, PROJ)

    cnt_all = _sc_counts(edges[0][1], edges[1][1], edges[2][1], edges[3][1])
    cnts = [cnt_all[t] for t in range(4)]

    tile8 = lambda v: jnp.tile(v[None, :], (8, 1))
    hu, hi = _tc_proj(gu, gi, Wp_u, tile8(bp_u), Wp_i, tile8(bp_i))

    srcs = (hu, hi, hi, hu)
    aggs = [_sc_agg(srcs[t], edges[t][0], edges[t][1]) for t in range(4)]
    wb_0 = jnp.tile(Wb[0][:, None, :], (1, 8, 1))
    hu, hi, ht = _tc_layer(aggs, cnts, hu, hi, None, Wl[0], Wr[0], wb_0)

    a2 = _sc_agg(hi, edges[2][0], edges[2][1])
    a3 = _sc_agg(hu, edges[3][0], edges[3][1])
    wb_1 = jnp.tile(Wb[1, 2:4][:, None, :], (1, 8, 1))
    return _tc_last(a2, a3, cnts[2], cnts[3], ht,
                    Wl[1, 2:4], Wr[1, 2:4], wb_1, out_W, tile8(out_b))
